# Initial kernel scaffold; baseline (speedup 1.0000x reference)
import jax
import jax.numpy as jnp
from jax.experimental import pallas as pl


def kernel(ts, theta, g0, g1, g2, g3):
    raise NotImplementedError("write your pallas kernel here")


# speedup vs baseline: 4.0631x; 4.0631x over previous
import functools
import math

import jax
import jax.numpy as jnp
from jax import lax
from jax.experimental import pallas as pl
from jax.experimental.pallas import tpu as pltpu
from jax.experimental.pallas import tpu_sc as plsc

B, N = 16, 8192
DIM = 32
PTS = B * N
ODIM = 128

NC, NS, LANES = 2, 16, 16
NW = NC * NS
PPW = PTS // NW
CH = 128
NCHUNK = PPW // CH

LEVELS = ((64, 256), (128, 512), (256, 1024), (512, 2048))

_PI = math.pi
_TWO_PI = 2.0 * math.pi


def _sc_body(ts_h, th_h, t0, t1, t2, t3, out_h,
             ts_v, th_v, idx4, w4, b4, out_v, sem):
    tabs = (t0, t1, t2, t3)
    wid = lax.axis_index("s") * NC + lax.axis_index("c")

    @pl.loop(0, NCHUNK)
    def _chunk(ci):
        base = wid * PPW + ci * CH
        pltpu.sync_copy(ts_h.at[pl.ds(base, CH)], ts_v)
        pltpu.sync_copy(th_h.at[pl.ds(base, CH)], th_v)

        for l, (H, W) in enumerate(LEVELS):
            tab = tabs[l]

            @pl.loop(0, CH // LANES)
            def _widx(i):
                s = i * LANES
                t16 = ts_v[pl.ds(s, LANES)]
                th16 = th_v[pl.ds(s, LANES)]
                thw = (th16 + _PI) / _TWO_PI
                ti = thw.astype(jnp.int32)
                tf = ti.astype(jnp.float32)
                fl = jnp.where(tf > thw, tf - 1.0, tf)
                frac = thw - fl
                gx = 2.0 * frac - 1.0
                gy = jnp.clip(t16, -1.0, 1.0)
                x = (gx + 1.0) * 0.5 * (W - 1)
                y = (gy + 1.0) * 0.5 * (H - 1)
                x = jnp.clip(x, 0.0, W - 1.0)
                y = jnp.clip(y, 0.0, H - 1.0)
                x0i = x.astype(jnp.int32)
                y0i = y.astype(jnp.int32)
                wx = x - x0i.astype(jnp.float32)
                wy = y - y0i.astype(jnp.float32)
                x1i = jnp.minimum(x0i + 1, W - 1)
                y1i = jnp.minimum(y0i + 1, H - 1)
                r0 = y0i * W
                r1 = y1i * W
                idx4[0, pl.ds(s, LANES)] = r0 + x0i
                idx4[1, pl.ds(s, LANES)] = r0 + x1i
                idx4[2, pl.ds(s, LANES)] = r1 + x0i
                idx4[3, pl.ds(s, LANES)] = r1 + x1i
                u = 1.0 - wx
                v = 1.0 - wy
                w4[0, pl.ds(s, LANES)] = u * v
                w4[1, pl.ds(s, LANES)] = wx * v
                w4[2, pl.ds(s, LANES)] = u * wy
                w4[3, pl.ds(s, LANES)] = wx * wy

            descs = [pltpu.async_copy(tab.at[idx4.at[k]], b4.at[k], sem)
                     for k in range(4)]
            for d in descs:
                d.wait()

            @pl.loop(0, CH // LANES)
            def _comb(i):
                s = i * LANES
                wv0 = w4[0, pl.ds(s, LANES)]
                wv1 = w4[1, pl.ds(s, LANES)]
                wv2 = w4[2, pl.ds(s, LANES)]
                wv3 = w4[3, pl.ds(s, LANES)]
                for j in range(LANES):
                    p = s + j
                    a0, a1, a2, a3 = wv0[j], wv1[j], wv2[j], wv3[j]
                    for half in (0, LANES):
                        acc = (b4[0, p, pl.ds(half, LANES)] * a0
                               + b4[1, p, pl.ds(half, LANES)] * a1
                               + b4[2, p, pl.ds(half, LANES)] * a2
                               + b4[3, p, pl.ds(half, LANES)] * a3)
                        out_v[p, pl.ds(l * DIM + half, LANES)] = acc

        pltpu.sync_copy(out_v, out_h.at[pl.ds(base, CH)])


@jax.jit
def _run(tsf, thf, tabs):
    mesh = plsc.VectorSubcoreMesh(core_axis_name="c", subcore_axis_name="s",
                                  num_cores=NC, num_subcores=NS)
    k = pl.kernel(
        _sc_body,
        out_type=jax.ShapeDtypeStruct((PTS, ODIM), jnp.float32),
        mesh=mesh,
        scratch_types=[
            pltpu.VMEM((CH,), jnp.float32),
            pltpu.VMEM((CH,), jnp.float32),
            pltpu.VMEM((4, CH), jnp.int32),
            pltpu.VMEM((4, CH), jnp.float32),
            pltpu.VMEM((4, CH, DIM), jnp.float32),
            pltpu.VMEM((CH, ODIM), jnp.float32),
            pltpu.SemaphoreType.DMA,
        ],
        compiler_params=pltpu.CompilerParams(use_tc_tiling_on_sc=False),
        name="curvetheta_multires_grid_sample",
    )
    return k(tsf, thf, *tabs)


def kernel(ts, theta, g0, g1, g2, g3):
    tabs = tuple(g[0].reshape(DIM, -1).T for g in (g0, g1, g2, g3))
    out = _run(ts.reshape(-1), theta.reshape(-1), tabs)
    return out.reshape(B, N, ODIM)
